# baseline (speedup 1.0000x reference)
import functools

import jax
import jax.numpy as jnp
from jax import lax
from jax.experimental import pallas as pl
from jax.experimental.pallas import tpu as pltpu
from jax.experimental.pallas import tpu_sc as plsc

B, T, C = 16, 2048, 512
NC, NS, L = 2, 16, 16
CW = C // NC
K = 64
NCHUNK = T // K
NBUF = 2
NV = CW // L
G = 8
NG = NV // G


@functools.partial(
    pl.kernel,
    out_type=jax.ShapeDtypeStruct((B, T, C // 128, 2, 128), jnp.float32),
    mesh=plsc.VectorSubcoreMesh(core_axis_name="c", subcore_axis_name="s"),
    compiler_params=pltpu.CompilerParams(needs_layout_passes=False),
    scratch_types=[
        pltpu.VMEM((NBUF, K, CW), jnp.float32),
        pltpu.VMEM((NBUF, K, CW // 128, 2, 128), jnp.float32),
        pltpu.VMEM((CW,), jnp.float32),
        pltpu.VMEM((CW,), jnp.float32),
        pltpu.VMEM((2, CW), jnp.float32),
        pltpu.SemaphoreType.DMA,
        pltpu.SemaphoreType.DMA,
        pltpu.SemaphoreType.DMA,
        pltpu.SemaphoreType.DMA,
    ],
)
def _recurrent_sc(x_hbm, ar_hbm, ai_hbm, out_hbm, in_v, out_v, ar_v, ai_v,
                  st_v, sem_in0, sem_in1, sem_out0, sem_out1):
    cid = lax.axis_index("c")
    sid = lax.axis_index("s")
    b = sid
    c0 = cid * CW
    sems_in = (sem_in0, sem_in1)
    sems_out = (sem_out0, sem_out1)

    pltpu.sync_copy(ar_hbm.at[pl.ds(c0, CW)], ar_v)
    pltpu.sync_copy(ai_hbm.at[pl.ds(c0, CW)], ai_v)

    zf = jnp.zeros((L,), jnp.float32)
    for j in range(NV):
        st_v[0, pl.ds(j * L, L)] = zf
        st_v[1, pl.ds(j * L, L)] = zf

    def in_copy(chunk, buf):
        return pltpu.make_async_copy(
            x_hbm.at[b, pl.ds(chunk * K, K), pl.ds(c0, CW)],
            in_v.at[buf], sems_in[buf])

    def out_copies(chunk, buf):
        return [
            pltpu.make_async_copy(
                out_v.at[buf],
                out_hbm.at[b, pl.ds(chunk * K, K), pl.ds(cid * (CW // 128), CW // 128)],
                sems_out[buf])
        ]

    def compute(buf):
        for g in range(NG):
            base = g * G * L
            sr0 = tuple(st_v[0, pl.ds(base + j * L, L)] for j in range(G))
            si0 = tuple(st_v[1, pl.ds(base + j * L, L)] for j in range(G))
            ar = [ar_v[pl.ds(base + j * L, L)] for j in range(G)]
            ai = [ai_v[pl.ds(base + j * L, L)] for j in range(G)]

            def tbody(t, carry, ar=ar, ai=ai, base=base):
                sr, si = carry
                nsr, nsi = [], []
                for j in range(G):
                    x = in_v[buf, t, pl.ds(base + j * L, L)]
                    nr = sr[j] * ar[j] - si[j] * ai[j] + x
                    ni = sr[j] * ai[j] + si[j] * ar[j]
                    c_off = base + j * L
                    out_v[buf, t, c_off // 128, 0, pl.ds(c_off % 128, L)] = nr
                    out_v[buf, t, c_off // 128, 1, pl.ds(c_off % 128, L)] = ni
                    nsr.append(nr)
                    nsi.append(ni)
                return (tuple(nsr), tuple(nsi))

            srf, sif = lax.fori_loop(0, K, tbody, (sr0, si0), unroll=1)
            for j in range(G):
                st_v[0, pl.ds(base + j * L, L)] = srf[j]
                st_v[1, pl.ds(base + j * L, L)] = sif[j]

    in_copy(0, 0).start()
    in_copy(1, 1).start()

    def outer(g2, _):
        for buf in range(NBUF):
            chunk = g2 * NBUF + buf
            in_copy(chunk, buf).wait()

            @pl.when(g2 >= 1)
            def _():
                for c_ in out_copies(chunk - NBUF, buf):
                    c_.wait()

            compute(buf)
            for c_ in out_copies(chunk, buf):
                c_.start()

            @pl.when(g2 < NCHUNK // NBUF - 1)
            def _():
                in_copy(chunk + NBUF, buf).start()
        return 0

    lax.fori_loop(0, NCHUNK // NBUF, outer, 0)
    for c_ in out_copies(NCHUNK - NBUF, 0):
        c_.wait()
    for c_ in out_copies(NCHUNK - 1, 1):
        c_.wait()


def kernel(inputs, A_real, A_imag):
    out = _recurrent_sc(inputs, A_real, A_imag)
    out = jnp.transpose(out, (0, 1, 2, 4, 3))
    return out.reshape(B, T, C, 2)


# speedup vs baseline: 1.8659x; 1.8659x over previous
import functools

import jax
import jax.numpy as jnp
from jax import lax
from jax.experimental import pallas as pl
from jax.experimental.pallas import tpu as pltpu
from jax.experimental.pallas import tpu_sc as plsc

B, T, C = 16, 2048, 512
NC, NS, L = 2, 16, 16
CW = C // NC
K = 64
NCHUNK = T // K
NBUF = 2
NV = CW // L
G = 8
NG = NV // G


@functools.partial(
    pl.kernel,
    out_type=jax.ShapeDtypeStruct((B, T, 2, C), jnp.float32),
    mesh=plsc.VectorSubcoreMesh(core_axis_name="c", subcore_axis_name="s"),
    compiler_params=pltpu.CompilerParams(needs_layout_passes=False),
    scratch_types=[
        pltpu.VMEM((NBUF, K, CW), jnp.float32),
        pltpu.VMEM((NBUF, 2, K, CW), jnp.float32),
        pltpu.VMEM((CW,), jnp.float32),
        pltpu.VMEM((CW,), jnp.float32),
        pltpu.VMEM((2, CW), jnp.float32),
        pltpu.SemaphoreType.DMA,
        pltpu.SemaphoreType.DMA,
        pltpu.SemaphoreType.DMA,
        pltpu.SemaphoreType.DMA,
    ],
)
def _recurrent_sc(x_hbm, ar_hbm, ai_hbm, out_hbm, in_v, out_v, ar_v, ai_v,
                  st_v, sem_in0, sem_in1, sem_out0, sem_out1):
    cid = lax.axis_index("c")
    sid = lax.axis_index("s")
    b = sid
    c0 = cid * CW
    sems_in = (sem_in0, sem_in1)
    sems_out = (sem_out0, sem_out1)

    pltpu.sync_copy(ar_hbm.at[pl.ds(c0, CW)], ar_v)
    pltpu.sync_copy(ai_hbm.at[pl.ds(c0, CW)], ai_v)

    zf = jnp.zeros((L,), jnp.float32)
    for j in range(NV):
        st_v[0, pl.ds(j * L, L)] = zf
        st_v[1, pl.ds(j * L, L)] = zf

    def in_copy(chunk, buf):
        return pltpu.make_async_copy(
            x_hbm.at[b, pl.ds(chunk * K, K), pl.ds(c0, CW)],
            in_v.at[buf], sems_in[buf])

    def out_copies(chunk, buf):
        H = K // 2
        return [
            pltpu.make_async_copy(
                out_v.at[buf, p, pl.ds(h * H, H)],
                out_hbm.at[b, pl.ds(chunk * K + h * H, H), p, pl.ds(c0, CW)],
                sems_out[buf])
            for p in range(2) for h in range(2)
        ]

    def compute(buf):
        for g in range(NG):
            base = g * G * L
            sr0 = tuple(st_v[0, pl.ds(base + j * L, L)] for j in range(G))
            si0 = tuple(st_v[1, pl.ds(base + j * L, L)] for j in range(G))
            ar = [ar_v[pl.ds(base + j * L, L)] for j in range(G)]
            ai = [ai_v[pl.ds(base + j * L, L)] for j in range(G)]

            def tbody(t, carry, ar=ar, ai=ai, base=base):
                sr, si = carry
                nsr, nsi = [], []
                for j in range(G):
                    x = in_v[buf, t, pl.ds(base + j * L, L)]
                    nr = sr[j] * ar[j] - si[j] * ai[j] + x
                    ni = sr[j] * ai[j] + si[j] * ar[j]
                    out_v[buf, 0, t, pl.ds(base + j * L, L)] = nr
                    out_v[buf, 1, t, pl.ds(base + j * L, L)] = ni
                    nsr.append(nr)
                    nsi.append(ni)
                return (tuple(nsr), tuple(nsi))

            srf, sif = lax.fori_loop(0, K, tbody, (sr0, si0), unroll=1)
            for j in range(G):
                st_v[0, pl.ds(base + j * L, L)] = srf[j]
                st_v[1, pl.ds(base + j * L, L)] = sif[j]

    in_copy(0, 0).start()
    in_copy(1, 1).start()

    def outer(g2, _):
        for buf in range(NBUF):
            chunk = g2 * NBUF + buf
            in_copy(chunk, buf).wait()

            @pl.when(g2 >= 1)
            def _():
                for c_ in out_copies(chunk - NBUF, buf):
                    c_.wait()

            compute(buf)
            for c_ in out_copies(chunk, buf):
                c_.start()

            @pl.when(g2 < NCHUNK // NBUF - 1)
            def _():
                in_copy(chunk + NBUF, buf).start()
        return 0

    lax.fori_loop(0, NCHUNK // NBUF, outer, 0)
    for c_ in out_copies(NCHUNK - NBUF, 0):
        c_.wait()
    for c_ in out_copies(NCHUNK - 1, 1):
        c_.wait()


def kernel(inputs, A_real, A_imag):
    out = _recurrent_sc(inputs, A_real, A_imag)
    return jnp.transpose(out, (0, 1, 3, 2))
